# baseline (speedup 1.0000x reference)
import functools

import jax
import jax.numpy as jnp
from jax import lax
from jax.experimental import pallas as pl
from jax.experimental.pallas import tpu as pltpu
from jax.experimental.pallas import tpu_sc as plsc

VOCAB = 100000
DIM = 64
CHUNK = 128
NBUF = 10
R = DIM // 8


def _make_lookup(batch: int, seq: int):
  info = plsc.get_sparse_core_info()
  nc, ns = info.num_cores, info.num_subcores
  nw = nc * ns
  assert batch % (nw * CHUNK) == 0 or batch == nw * CHUNK
  nchunks = batch // CHUNK
  assert nchunks == nw
  steps = seq

  mesh = plsc.VectorSubcoreMesh(core_axis_name="c", subcore_axis_name="s")

  @functools.partial(
      pl.kernel,
      mesh=mesh,
      compiler_params=pltpu.CompilerParams(use_tc_tiling_on_sc=False, needs_layout_passes=False),
      out_type=jax.ShapeDtypeStruct((steps, R, nchunks, 8, CHUNK), jnp.float32),
      scratch_types=[
          pltpu.VMEM((steps, CHUNK), jnp.int32),
          pltpu.VMEM((NBUF * CHUNK, DIM), jnp.float32),
          pltpu.VMEM((2, R, 1, 8, CHUNK), jnp.float32),
          pltpu.SemaphoreType.DMA,
          pltpu.SemaphoreType.DMA,
      ],
  )
  def lookup_kernel(xt_hbm, table_hbm, out_hbm, idx_v, rows_v, trans_v, gsem, osem):
    w = lax.axis_index("s") * nc + lax.axis_index("c")
    pltpu.sync_copy(xt_hbm.at[:, pl.ds(w * CHUNK, CHUNK)], idx_v)

    iota16 = lax.iota(jnp.int32, 16)

    def start_gather(s):
      buf = lax.rem(s, NBUF)
      pltpu.async_copy(
          table_hbm.at[idx_v.at[s]],
          rows_v.at[pl.ds(buf * CHUNK, CHUNK)],
          gsem,
      )

    def wait_gather():
      pltpu.make_async_copy(
          table_hbm.at[pl.ds(0, CHUNK)],
          rows_v.at[pl.ds(0, CHUNK)],
          gsem,
      ).wait()

    def wait_out():
      pltpu.make_async_copy(
          trans_v.at[0],
          out_hbm.at[pl.ds(0, 1), :, pl.ds(0, 1)],
          osem,
      ).wait()

    rvecs = [2 * k + iota16 // 8 for k in range(DIM // 16)]
    ddvec = iota16 % 8
    zero16 = jnp.zeros((16,), jnp.int32)
    perms = [(iota16 + c) % 16 for c in range(16)]
    cidxs = [16 * k + iota16 for k in range(DIM // 16)]

    def transpose_and_store(s):
      buf = lax.rem(s, NBUF)
      tb = lax.rem(s, 2)
      base = buf * CHUNK
      tbv = jnp.broadcast_to(tb, (16,))
      basev = jnp.broadcast_to(base, (16,))

      for g in range(CHUNK // 16):
        for c in range(16):
          bbv = 16 * g + perms[c]
          ridx = basev + bbv
          for k in range(DIM // 16):
            v = plsc.load_gather(rows_v, [ridx, cidxs[k]])
            plsc.store_scatter(trans_v, [tbv, rvecs[k], zero16, ddvec, bbv], v)
      pltpu.async_copy(
          trans_v.at[tb],
          out_hbm.at[s, :, pl.ds(w, 1)],
          osem,
      )

    for s in range(NBUF):
      start_gather(s)

    def step_head(s, carry):
      wait_gather()
      transpose_and_store(s)
      start_gather(s + NBUF)
      return carry

    def step_main(s, carry):
      wait_out()
      wait_gather()
      transpose_and_store(s)
      start_gather(s + NBUF)
      return carry

    def step_tail(s, carry):
      wait_out()
      wait_gather()
      transpose_and_store(s)
      return carry

    lax.fori_loop(0, 2, step_head, 0)
    lax.fori_loop(2, steps - NBUF, step_main, 0)
    lax.fori_loop(steps - NBUF, steps, step_tail, 0)
    wait_out()
    wait_out()

  return lookup_kernel


def kernel(x, table):
  b, s = x.shape
  xt = jnp.transpose(x)
  out5 = _make_lookup(b, s)(xt, table)
  return out5.transpose(2, 4, 0, 1, 3).reshape(b, s, DIM)


# speedup vs baseline: 1.4449x; 1.4449x over previous
import functools

import jax
import jax.numpy as jnp
from jax import lax
from jax.experimental import pallas as pl
from jax.experimental.pallas import tpu as pltpu
from jax.experimental.pallas import tpu_sc as plsc

VOCAB = 100000
DIM = 64
CHUNK = 128
NBUF = 10
R = DIM // 8


def _make_lookup(batch: int, seq: int):
  info = plsc.get_sparse_core_info()
  nc, ns = info.num_cores, info.num_subcores
  nw = nc * ns
  assert batch % (nw * CHUNK) == 0 or batch == nw * CHUNK
  nchunks = batch // CHUNK
  assert nchunks == nw
  steps = seq

  mesh = plsc.VectorSubcoreMesh(core_axis_name="c", subcore_axis_name="s")

  @functools.partial(
      pl.kernel,
      mesh=mesh,
      compiler_params=pltpu.CompilerParams(use_tc_tiling_on_sc=False, needs_layout_passes=False),
      out_type=jax.ShapeDtypeStruct((steps, R, nchunks, 8, CHUNK), jnp.float32),
      scratch_types=[
          pltpu.VMEM((steps, CHUNK), jnp.int32),
          pltpu.VMEM((NBUF * CHUNK, DIM), jnp.float32),
          pltpu.VMEM((2, R, 1, 8, CHUNK), jnp.float32),
          pltpu.SemaphoreType.DMA,
          pltpu.SemaphoreType.DMA,
      ],
  )
  def lookup_kernel(xt_hbm, table_hbm, out_hbm, idx_v, rows_v, trans_v, gsem, osem):
    w = lax.axis_index("s") * nc + lax.axis_index("c")
    pltpu.sync_copy(xt_hbm.at[:, pl.ds(w * CHUNK, CHUNK)], idx_v)

    iota16 = lax.iota(jnp.int32, 16)

    def start_gather(s):
      buf = lax.rem(s, NBUF)
      pltpu.async_copy(
          table_hbm.at[idx_v.at[s]],
          rows_v.at[pl.ds(buf * CHUNK, CHUNK)],
          gsem,
      )

    def wait_gather():
      pltpu.make_async_copy(
          table_hbm.at[pl.ds(0, CHUNK)],
          rows_v.at[pl.ds(0, CHUNK)],
          gsem,
      ).wait()

    def wait_out():
      pltpu.make_async_copy(
          trans_v.at[0],
          out_hbm.at[pl.ds(0, 1), :, pl.ds(0, 1)],
          osem,
      ).wait()

    rvecs = [2 * k + iota16 // 8 for k in range(DIM // 16)]
    ddvec = iota16 % 8
    zero16 = jnp.zeros((16,), jnp.int32)
    perms = [(iota16 + c) % 16 for c in range(16)]
    cidxs = [16 * k + iota16 for k in range(DIM // 16)]

    def transpose_and_store(s):
      buf = lax.rem(s, NBUF)
      tb = lax.rem(s, 2)
      base = buf * CHUNK
      tbv = jnp.broadcast_to(tb, (16,))
      basev = jnp.broadcast_to(base, (16,))

      def per_g(g, carry):
        g16 = jnp.broadcast_to(16 * g, (16,))
        for c in range(16):
          bbv = g16 + perms[c]
          ridx = basev + bbv
          for k in range(DIM // 16):
            v = plsc.load_gather(rows_v, [ridx, cidxs[k]])
            plsc.store_scatter(trans_v, [tbv, rvecs[k], zero16, ddvec, bbv], v)
        return carry

      lax.fori_loop(0, CHUNK // 16, per_g, 0)
      pltpu.async_copy(
          trans_v.at[tb],
          out_hbm.at[s, :, pl.ds(w, 1)],
          osem,
      )

    for s in range(NBUF):
      start_gather(s)

    def step_head(s, carry):
      wait_gather()
      transpose_and_store(s)
      start_gather(s + NBUF)
      return carry

    def step_main(s, carry):
      wait_out()
      wait_gather()
      transpose_and_store(s)
      start_gather(s + NBUF)
      return carry

    def step_tail(s, carry):
      wait_out()
      wait_gather()
      transpose_and_store(s)
      return carry

    lax.fori_loop(0, 2, step_head, 0)
    lax.fori_loop(2, steps - NBUF, step_main, 0)
    lax.fori_loop(steps - NBUF, steps, step_tail, 0)
    wait_out()
    wait_out()

  return lookup_kernel


def kernel(x, table):
  b, s = x.shape
  xt = jnp.transpose(x)
  out5 = _make_lookup(b, s)(xt, table)
  return out5.transpose(2, 4, 0, 1, 3).reshape(b, s, DIM)


# speedup vs baseline: 1.4504x; 1.0038x over previous
import functools

import jax
import jax.numpy as jnp
from jax import lax
from jax.experimental import pallas as pl
from jax.experimental.pallas import tpu as pltpu
from jax.experimental.pallas import tpu_sc as plsc

VOCAB = 100000
DIM = 64
CHUNK = 128
NBUF = 8
R = DIM // 8


def _make_lookup(batch: int, seq: int):
  info = plsc.get_sparse_core_info()
  nc, ns = info.num_cores, info.num_subcores
  nw = nc * ns
  assert batch % (nw * CHUNK) == 0 or batch == nw * CHUNK
  nchunks = batch // CHUNK
  assert nchunks == nw
  steps = seq

  mesh = plsc.VectorSubcoreMesh(core_axis_name="c", subcore_axis_name="s")

  @functools.partial(
      pl.kernel,
      mesh=mesh,
      compiler_params=pltpu.CompilerParams(use_tc_tiling_on_sc=False, needs_layout_passes=False),
      out_type=jax.ShapeDtypeStruct((steps, R, nchunks, 8, CHUNK), jnp.float32),
      scratch_types=[
          pltpu.VMEM((steps, CHUNK), jnp.int32),
          pltpu.VMEM((NBUF * CHUNK, DIM), jnp.float32),
          pltpu.VMEM((2, R, 1, 8, CHUNK), jnp.float32),
          pltpu.SemaphoreType.DMA,
          pltpu.SemaphoreType.DMA,
      ],
  )
  def lookup_kernel(xt_hbm, table_hbm, out_hbm, idx_v, rows_v, trans_v, gsem, osem):
    w = lax.axis_index("s") * nc + lax.axis_index("c")
    pltpu.sync_copy(xt_hbm.at[:, pl.ds(w * CHUNK, CHUNK)], idx_v)

    iota16 = lax.iota(jnp.int32, 16)

    def start_gather(s):
      buf = lax.rem(s, NBUF)
      pltpu.async_copy(
          table_hbm.at[idx_v.at[s]],
          rows_v.at[pl.ds(buf * CHUNK, CHUNK)],
          gsem,
      )

    def wait_gather():
      pltpu.make_async_copy(
          table_hbm.at[pl.ds(0, CHUNK)],
          rows_v.at[pl.ds(0, CHUNK)],
          gsem,
      ).wait()

    def wait_out():
      pltpu.make_async_copy(
          trans_v.at[0],
          out_hbm.at[pl.ds(0, 1), :, pl.ds(0, 1)],
          osem,
      ).wait()

    rvecs = [2 * k + iota16 // 8 for k in range(DIM // 16)]
    ddvec = iota16 % 8
    zero16 = jnp.zeros((16,), jnp.int32)
    perms = [(iota16 + c) % 16 for c in range(16)]
    cidxs = [16 * k + iota16 for k in range(DIM // 16)]

    def transpose_and_store(s):
      buf = lax.rem(s, NBUF)
      tb = lax.rem(s, 2)
      base = buf * CHUNK
      tbv = jnp.broadcast_to(tb, (16,))
      basev = jnp.broadcast_to(base, (16,))

      def per_g(g, carry):
        g16 = jnp.broadcast_to(16 * g, (16,))
        for c in range(16):
          bbv = g16 + perms[c]
          ridx = basev + bbv
          for k in range(DIM // 16):
            v = plsc.load_gather(rows_v, [ridx, cidxs[k]])
            plsc.store_scatter(trans_v, [tbv, rvecs[k], zero16, ddvec, bbv], v)
        return carry

      lax.fori_loop(0, CHUNK // 16, per_g, 0)
      pltpu.async_copy(
          trans_v.at[tb],
          out_hbm.at[s, :, pl.ds(w, 1)],
          osem,
      )

    for s in range(NBUF):
      start_gather(s)

    def step_head(s, carry):
      wait_gather()
      transpose_and_store(s)
      start_gather(s + NBUF)
      return carry

    def step_main(s, carry):
      wait_out()
      wait_gather()
      transpose_and_store(s)
      start_gather(s + NBUF)
      return carry

    def step_tail(s, carry):
      wait_out()
      wait_gather()
      transpose_and_store(s)
      return carry

    lax.fori_loop(0, 2, step_head, 0)
    lax.fori_loop(2, steps - NBUF, step_main, 0)
    lax.fori_loop(steps - NBUF, steps, step_tail, 0)
    wait_out()
    wait_out()

  return lookup_kernel


def kernel(x, table):
  b, s = x.shape
  xt = jnp.transpose(x)
  out5 = _make_lookup(b, s)(xt, table)
  return out5.transpose(2, 4, 0, 1, 3).reshape(b, s, DIM)
